# baseline (speedup 1.0000x reference)
import functools

import jax
import jax.numpy as jnp
from jax import lax
from jax.experimental import pallas as pl
from jax.experimental.pallas import tpu as pltpu
from jax.experimental.pallas import tpu_sc as plsc

NUM_HEROES = 1000000
EMBED_DIM = 32
BATCH = 16384
SEQ = 10

_info = plsc.get_sparse_core_info()
NC, NS, NL = _info.num_cores, _info.num_subcores, _info.num_lanes
NW = NC * NS
NBT = BATCH // 128
BT_PER_W = NBT // NW
SEQ_PAD = 16
TBLK = NUM_HEROES // 4
OUT_ROWS = SEQ * BATCH * EMBED_DIM // 128


def _body(q4_hbm, r4_hbm, tw_hbm, out2, qix, rem_v, bg0, bg1, bc0, bc1,
          g0, g1, w0, w1):
    wid = lax.axis_index("s") * NC + lax.axis_index("c")
    bgs = (bg0, bg1)
    bcs = (bc0, bc1)
    gsems = (g0, g1)
    wsems = (w0, w1)

    for c in range(BT_PER_W):
        bt = BT_PER_W * wid + c
        pltpu.sync_copy(q4_hbm.at[0, bt], qix.at[c, pl.ds(0, 8)])
        pltpu.sync_copy(q4_hbm.at[1, bt, pl.ds(0, 2)], qix.at[c, pl.ds(8, 2)])
        pltpu.sync_copy(r4_hbm.at[0, bt], rem_v.at[c, pl.ds(0, 8)])
        pltpu.sync_copy(r4_hbm.at[1, bt, pl.ds(0, 2)], rem_v.at[c, pl.ds(8, 2)])

    def gather(c, s, b):
        return pltpu.make_async_copy(tw_hbm.at[qix.at[c, s]], bgs[b], gsems[b])

    def write(c, s, b):
        row = s * (BATCH * EMBED_DIM // 128) + (BT_PER_W * wid + c) * 32
        return pltpu.make_async_copy(bcs[b], out2.at[pl.ds(row, 32)], wsems[b])

    def chunk(c, carry):
        gather(c, 0, 0).start()
        for s in range(SEQ):
            b = s % 2
            gather(c, s, b).wait()
            if s + 1 < SEQ:
                gather(c, s + 1, 1 - b).start()
            if s >= 2:
                write(c, s - 2, b).wait()
            for g in range(128 // NL):
                rv = rem_v[c, s, pl.ds(g * NL, NL)]
                for jj in range(NL):
                    j = g * NL + jj
                    off = rv[jj]
                    lo = bgs[b][j, pl.ds(off, NL)]
                    hi = bgs[b][j, pl.ds(off + NL, NL)]
                    bcs[b][j // 4, pl.ds((j % 4) * EMBED_DIM, NL)] = lo
                    bcs[b][j // 4, pl.ds((j % 4) * EMBED_DIM + NL, NL)] = hi
            write(c, s, b).start()
        write(c, SEQ - 2, 0).wait()
        write(c, SEQ - 1, 1).wait()
        return carry

    lax.fori_loop(0, BT_PER_W, chunk, 0)


@jax.jit
def kernel(x, table):
    def to4(a):
        ap = jnp.pad(a.T, ((0, SEQ_PAD - SEQ), (0, 0)))
        return ap.reshape(2, 8, NBT, 128).transpose(0, 2, 1, 3)

    q4 = to4(x >> 2)
    r4 = to4((x & 3) * EMBED_DIM)

    run = pl.kernel(
        _body,
        out_type=jax.ShapeDtypeStruct((OUT_ROWS, 128), jnp.float32),
        mesh=plsc.VectorSubcoreMesh(core_axis_name="c", subcore_axis_name="s"),
        scratch_types=[
            pltpu.VMEM((BT_PER_W, SEQ, 128), jnp.int32),
            pltpu.VMEM((BT_PER_W, SEQ, 128), jnp.int32),
            pltpu.VMEM((128, 128), jnp.float32),
            pltpu.VMEM((128, 128), jnp.float32),
            pltpu.VMEM((32, 128), jnp.float32),
            pltpu.VMEM((32, 128), jnp.float32),
            pltpu.SemaphoreType.DMA,
            pltpu.SemaphoreType.DMA,
            pltpu.SemaphoreType.DMA,
            pltpu.SemaphoreType.DMA,
        ],
        compiler_params=pltpu.CompilerParams(use_tc_tiling_on_sc=False),
    )
    t1 = lax.optimization_barrier(table.reshape(NUM_HEROES * EMBED_DIM))
    out2 = run(q4, r4, t1.reshape(TBLK, 128))
    return out2.reshape(SEQ, BATCH, EMBED_DIM).transpose(1, 0, 2)
